# Initial kernel scaffold; baseline (speedup 1.0000x reference)
import jax
import jax.numpy as jnp
from jax.experimental import pallas as pl


def kernel(h, edge_index, edge_type, weight, w_comp, self_loop_weight):
    raise NotImplementedError("write your pallas kernel here")


# speedup vs baseline: 2.6993x; 2.6993x over previous
import functools

import jax
import jax.numpy as jnp
from jax import lax
from jax.experimental import pallas as pl
from jax.experimental.pallas import tpu as pltpu
from jax.experimental.pallas import tpu_sc as plsc


def _hrel_table(h, weight, w_comp):
    N, D = h.shape
    B, _, O = weight.shape
    R = w_comp.shape[0]
    BN = 2000
    NB = N // BN

    def body(wc_ref, w_ref, h_ref, out_ref):
        acc = w_ref[0] * wc_ref[0, 0, 0]
        for b in range(1, B):
            acc = acc + w_ref[b] * wc_ref[0, 0, b]
        out_ref[...] = jnp.dot(h_ref[...], acc,
                               preferred_element_type=jnp.float32)

    return pl.pallas_call(
        body,
        grid=(NB, R),
        in_specs=[
            pl.BlockSpec((1, 1, B), lambda i, r: (r, 0, 0)),
            pl.BlockSpec((B, D, O), lambda i, r: (0, 0, 0)),
            pl.BlockSpec((BN, D), lambda i, r: (i, 0)),
        ],
        out_specs=pl.BlockSpec((BN, O), lambda i, r: (r * NB + i, 0)),
        out_shape=jax.ShapeDtypeStruct((R * N, O), jnp.float32),
    )(w_comp.reshape(R, 1, B), weight, h)


def _sc_gather_scatter(hrel, srcr, etr, dstr, N, N_pad, O):
    info = plsc.get_sparse_core_info()
    NC, NS = info.num_cores, info.num_subcores
    NW = NC * NS
    _, NSTEPS, CH = srcr.shape
    RPT = N_pad // NS

    mesh = plsc.VectorSubcoreMesh(core_axis_name="c", subcore_axis_name="s")

    @functools.partial(
        pl.kernel,
        mesh=mesh,
        out_type=jax.ShapeDtypeStruct((NC, N_pad, O), jnp.float32),
        scratch_types=[
            pltpu.VMEM((NSTEPS, CH), jnp.int32),
            pltpu.VMEM((NSTEPS, CH), jnp.int32),
            pltpu.VMEM((NSTEPS, CH), jnp.int32),
            pltpu.VMEM((CH, O), jnp.float32),
            pltpu.VMEM_SHARED((N_pad, O), jnp.float32),
            pltpu.SemaphoreType.DMA,
        ],
    )
    def sc_body(hrel_hbm, src_hbm, et_hbm, dst_hbm, out_hbm,
                idx_v, et_v, dst_v, rows_v, acc, sem):
        cid = lax.axis_index("c")
        sid = lax.axis_index("s")
        wid = cid * NS + sid

        pltpu.sync_copy(src_hbm.at[wid], idx_v)
        pltpu.sync_copy(et_hbm.at[wid], et_v)
        pltpu.sync_copy(dst_hbm.at[wid], dst_v)

        def gix(i, carry):
            for j in range(CH // 16):
                sl = pl.ds(j * 16, 16)
                idx_v[i, sl] = et_v[i, sl] * N + idx_v[i, sl]
            return carry
        lax.fori_loop(0, NSTEPS, gix, 0)

        def zr(i, carry):
            for j in range(O // 16):
                rows_v[i, pl.ds(j * 16, 16)] = jnp.zeros((16,), jnp.float32)
            return carry
        lax.fori_loop(0, CH, zr, 0)
        base = sid * RPT
        nfull = RPT // CH
        for k in range(nfull):
            pltpu.sync_copy(rows_v, acc.at[pl.ds(base + k * CH, CH)])
        rem = RPT - nfull * CH
        if rem:
            pltpu.sync_copy(rows_v.at[pl.ds(0, rem)],
                            acc.at[pl.ds(base + nfull * CH, rem)])
        plsc.subcore_barrier()

        def step(s, carry):
            pltpu.async_copy(hrel_hbm.at[idx_v.at[s]], rows_v, sem).wait()
            pltpu.sync_copy(rows_v, acc.at[dst_v.at[s]], add=True)
            return carry
        lax.fori_loop(0, NSTEPS, step, 0)
        plsc.subcore_barrier()

        pltpu.sync_copy(acc.at[pl.ds(base, RPT)],
                        out_hbm.at[cid, pl.ds(base, RPT)])

    return sc_body(hrel, srcr, etr, dstr)


def _combine(parts, h, self_loop_weight):
    N, D = h.shape
    O = self_loop_weight.shape[1]
    BN = 2000

    def body(p_ref, h_ref, sw_ref, out_ref):
        out_ref[...] = (p_ref[0] + p_ref[1]
                        + jnp.dot(h_ref[...], sw_ref[...],
                                  preferred_element_type=jnp.float32))

    return pl.pallas_call(
        body,
        grid=(N // BN,),
        in_specs=[
            pl.BlockSpec((2, BN, O), lambda i: (0, i, 0)),
            pl.BlockSpec((BN, D), lambda i: (i, 0)),
            pl.BlockSpec((D, O), lambda i: (0, 0)),
        ],
        out_specs=pl.BlockSpec((BN, O), lambda i: (i, 0)),
        out_shape=jax.ShapeDtypeStruct((N, O), jnp.float32),
    )(parts, h, self_loop_weight)


def kernel(h, edge_index, edge_type, weight, w_comp, self_loop_weight):
    N, D = h.shape
    O = weight.shape[2]
    E = edge_type.shape[0]

    hrel = _hrel_table(h, weight, w_comp)

    NW = 32
    CH = 128
    T = -(-E // NW)
    T_pad = -(-T // CH) * CH
    pad = NW * T_pad - E
    NSTEPS = T_pad // CH
    src = edge_index[0].astype(jnp.int32)
    dst = edge_index[1].astype(jnp.int32)
    et = edge_type.astype(jnp.int32)
    srcr = jnp.concatenate([src, jnp.zeros((pad,), jnp.int32)])
    etr = jnp.concatenate([et, jnp.zeros((pad,), jnp.int32)])
    dstr = jnp.concatenate([dst, jnp.full((pad,), N, jnp.int32)])
    srcr = srcr.reshape(NW, NSTEPS, CH)
    etr = etr.reshape(NW, NSTEPS, CH)
    dstr = dstr.reshape(NW, NSTEPS, CH)
    N_pad = -(-(N + 1) // 128) * 128

    parts = _sc_gather_scatter(hrel, srcr, etr, dstr, N, N_pad, O)
    return _combine(parts, h, self_loop_weight)


# speedup vs baseline: 2.8771x; 1.0659x over previous
import functools

import jax
import jax.numpy as jnp
from jax import lax
from jax.experimental import pallas as pl
from jax.experimental.pallas import tpu as pltpu
from jax.experimental.pallas import tpu_sc as plsc


def _hrel_table(h, weight, w_comp):
    N, D = h.shape
    B, _, O = weight.shape
    R = w_comp.shape[0]
    BN = 2000
    NB = N // BN

    def body(wc_ref, w_ref, h_ref, out_ref):
        acc = w_ref[0] * wc_ref[0, 0, 0]
        for b in range(1, B):
            acc = acc + w_ref[b] * wc_ref[0, 0, b]
        out_ref[...] = jnp.dot(h_ref[...], acc,
                               preferred_element_type=jnp.float32)

    return pl.pallas_call(
        body,
        grid=(NB, R),
        in_specs=[
            pl.BlockSpec((1, 1, B), lambda i, r: (r, 0, 0)),
            pl.BlockSpec((B, D, O), lambda i, r: (0, 0, 0)),
            pl.BlockSpec((BN, D), lambda i, r: (i, 0)),
        ],
        out_specs=pl.BlockSpec((BN, O), lambda i, r: (r * NB + i, 0)),
        out_shape=jax.ShapeDtypeStruct((R * N, O), jnp.float32),
    )(w_comp.reshape(R, 1, B), weight, h)


def _sc_gather_scatter(hrel, gidxr, dstr, N_pad, O):
    info = plsc.get_sparse_core_info()
    NC, NS = info.num_cores, info.num_subcores
    _, NSTEPS, CH = gidxr.shape
    RPT = N_pad // NS

    mesh = plsc.VectorSubcoreMesh(core_axis_name="c", subcore_axis_name="s")

    @functools.partial(
        pl.kernel,
        mesh=mesh,
        out_type=jax.ShapeDtypeStruct((NC, N_pad, O), jnp.float32),
        scratch_types=[
            pltpu.VMEM((NSTEPS, CH), jnp.int32),
            pltpu.VMEM((2, 1, CH), jnp.int32),
            pltpu.VMEM((CH, O), jnp.float32),
            pltpu.VMEM((CH, O), jnp.float32),
            pltpu.VMEM_SHARED((N_pad, O), jnp.float32),
            pltpu.SemaphoreType.DMA,
            pltpu.SemaphoreType.DMA,
            pltpu.SemaphoreType.DMA,
            pltpu.SemaphoreType.DMA,
        ],
    )
    def sc_body(hrel_hbm, gidx_hbm, dst_hbm, out_hbm,
                idx_v, dst_b, rows_0, rows_1, acc, sem0, sem1, semd0, semd1):
        rows_v = rows_0
        cid = lax.axis_index("c")
        sid = lax.axis_index("s")
        wid = cid * NS + sid

        pltpu.sync_copy(gidx_hbm.at[wid], idx_v)

        def zr(i, carry):
            for j in range(O // 16):
                rows_v[i, pl.ds(j * 16, 16)] = jnp.zeros((16,), jnp.float32)
            return carry
        lax.fori_loop(0, CH, zr, 0)
        base = sid * RPT
        nfull = RPT // CH
        for k in range(nfull):
            pltpu.sync_copy(rows_v, acc.at[pl.ds(base + k * CH, CH)])
        rem = RPT - nfull * CH
        if rem:
            pltpu.sync_copy(rows_v.at[pl.ds(0, rem)],
                            acc.at[pl.ds(base + nfull * CH, rem)])
        plsc.subcore_barrier()

        bufs = (rows_0, rows_1)
        sems = (sem0, sem1)
        dsems = (semd0, semd1)

        def start_g(s, k):
            pltpu.async_copy(hrel_hbm.at[idx_v.at[s]], bufs[k], sems[k])
            pltpu.async_copy(dst_hbm.at[wid, s], dst_b.at[k], dsems[k])

        def wait_g(s, k):
            pltpu.make_async_copy(hrel_hbm.at[idx_v.at[s]], bufs[k],
                                  sems[k]).wait()
            pltpu.make_async_copy(dst_hbm.at[wid, s], dst_b.at[k],
                                  dsems[k]).wait()

        def scat(s, k):
            pltpu.sync_copy(bufs[k], acc.at[dst_b.at[k, 0]], add=True)

        npair = (NSTEPS - 1) // 2 if NSTEPS % 2 else (NSTEPS - 2) // 2
        start_g(0, 0)

        def pair(i, carry):
            s = 2 * i
            start_g(s + 1, 1)
            wait_g(s, 0)
            scat(s, 0)
            start_g(s + 2, 0)
            wait_g(s + 1, 1)
            scat(s + 1, 1)
            return carry
        lax.fori_loop(0, npair, pair, 0)
        if NSTEPS % 2:
            wait_g(NSTEPS - 1, 0)
            scat(NSTEPS - 1, 0)
        else:
            start_g(NSTEPS - 1, 1)
            wait_g(NSTEPS - 2, 0)
            scat(NSTEPS - 2, 0)
            wait_g(NSTEPS - 1, 1)
            scat(NSTEPS - 1, 1)
        plsc.subcore_barrier()

        pltpu.sync_copy(acc.at[pl.ds(base, RPT)],
                        out_hbm.at[cid, pl.ds(base, RPT)])

    return sc_body(hrel, gidxr, dstr)


def _gidx_table(srcp, etp, N):
    E_pad = srcp.shape[0]
    W = 512
    rows = E_pad // W

    def body(s_ref, t_ref, o_ref):
        o_ref[...] = t_ref[...] * N + s_ref[...]

    return pl.pallas_call(
        body,
        out_shape=jax.ShapeDtypeStruct((rows, W), jnp.int32),
    )(srcp.reshape(rows, W), etp.reshape(rows, W))


def _combine(parts, h, self_loop_weight):
    N, D = h.shape
    O = self_loop_weight.shape[1]
    BN = 2000

    def body(p_ref, h_ref, sw_ref, out_ref):
        out_ref[...] = (p_ref[0] + p_ref[1]
                        + jnp.dot(h_ref[...], sw_ref[...],
                                  preferred_element_type=jnp.float32))

    return pl.pallas_call(
        body,
        grid=(N // BN,),
        in_specs=[
            pl.BlockSpec((2, BN, O), lambda i: (0, i, 0)),
            pl.BlockSpec((BN, D), lambda i: (i, 0)),
            pl.BlockSpec((D, O), lambda i: (0, 0)),
        ],
        out_specs=pl.BlockSpec((BN, O), lambda i: (i, 0)),
        out_shape=jax.ShapeDtypeStruct((N, O), jnp.float32),
    )(parts, h, self_loop_weight)


def kernel(h, edge_index, edge_type, weight, w_comp, self_loop_weight):
    N, D = h.shape
    O = weight.shape[2]
    E = edge_type.shape[0]

    hrel = _hrel_table(h, weight, w_comp)

    NW = 32
    CH = 128
    T = -(-E // NW)
    T_pad = -(-T // CH) * CH
    pad = NW * T_pad - E
    NSTEPS = T_pad // CH
    src = edge_index[0].astype(jnp.int32)
    dst = edge_index[1].astype(jnp.int32)
    et = edge_type.astype(jnp.int32)
    srcp = jnp.concatenate([src, jnp.zeros((pad,), jnp.int32)])
    etp = jnp.concatenate([et, jnp.zeros((pad,), jnp.int32)])
    dstp = jnp.concatenate([dst, jnp.full((pad,), N, jnp.int32)])
    gidxr = _gidx_table(srcp, etp, N).reshape(NW, NSTEPS, CH)
    dstr = dstp.reshape(NW, NSTEPS, 1, CH)
    N_pad = -(-(N + 1) // 128) * 128

    parts = _sc_gather_scatter(hrel, gidxr, dstr, N_pad, O)
    return _combine(parts, h, self_loop_weight)


# speedup vs baseline: 4.5030x; 1.5651x over previous
import functools

import jax
import jax.numpy as jnp
from jax import lax
from jax.experimental import pallas as pl
from jax.experimental.pallas import tpu as pltpu
from jax.experimental.pallas import tpu_sc as plsc


def _hrel_table(h, weight, w_comp):
    N, D = h.shape
    B, _, O = weight.shape
    R = w_comp.shape[0]
    BN = 2000
    NB = N // BN

    def body(wc_ref, w_ref, h_ref, out_ref):
        acc = w_ref[0] * wc_ref[0, 0, 0]
        for b in range(1, B):
            acc = acc + w_ref[b] * wc_ref[0, 0, b]
        out_ref[...] = jnp.dot(h_ref[...], acc,
                               preferred_element_type=jnp.float32)

    return pl.pallas_call(
        body,
        grid=(NB, R),
        in_specs=[
            pl.BlockSpec((1, 1, B), lambda i, r: (r, 0, 0)),
            pl.BlockSpec((B, D, O), lambda i, r: (0, 0, 0)),
            pl.BlockSpec((BN, D), lambda i, r: (i, 0)),
        ],
        out_specs=pl.BlockSpec((BN, O), lambda i, r: (r * NB + i, 0)),
        out_shape=jax.ShapeDtypeStruct((R * N, O), jnp.float32),
    )(w_comp.reshape(R, 1, B), weight, h)


def _sc_gather_scatter(hrel, gidxr, dstr, N_pad, O):
    info = plsc.get_sparse_core_info()
    NC, NS = info.num_cores, info.num_subcores
    _, NSTEPS, CH = gidxr.shape
    RPT = N_pad // NS

    mesh = plsc.VectorSubcoreMesh(core_axis_name="c", subcore_axis_name="s")

    @functools.partial(
        pl.kernel,
        mesh=mesh,
        out_type=jax.ShapeDtypeStruct((NC, N_pad, O), jnp.float32),
        scratch_types=[
            pltpu.VMEM((NSTEPS, CH), jnp.int32),
            pltpu.VMEM((2, 1, CH), jnp.int32),
            pltpu.VMEM((CH, O), jnp.float32),
            pltpu.VMEM((CH, O), jnp.float32),
            pltpu.VMEM_SHARED((N_pad, O), jnp.float32),
            pltpu.SemaphoreType.DMA,
            pltpu.SemaphoreType.DMA,
            pltpu.SemaphoreType.DMA,
            pltpu.SemaphoreType.DMA,
        ],
    )
    def sc_body(hrel_hbm, gidx_hbm, dst_hbm, out_hbm,
                idx_v, dst_b, rows_0, rows_1, acc, sem0, sem1, semd0, semd1):
        rows_v = rows_0
        cid = lax.axis_index("c")
        sid = lax.axis_index("s")
        wid = cid * NS + sid

        pltpu.sync_copy(gidx_hbm.at[wid], idx_v)

        def zr(i, carry):
            for j in range(O // 16):
                rows_v[i, pl.ds(j * 16, 16)] = jnp.zeros((16,), jnp.float32)
            return carry
        lax.fori_loop(0, CH, zr, 0)
        base = sid * RPT
        nfull = RPT // CH
        for k in range(nfull):
            pltpu.sync_copy(rows_v, acc.at[pl.ds(base + k * CH, CH)])
        rem = RPT - nfull * CH
        if rem:
            pltpu.sync_copy(rows_v.at[pl.ds(0, rem)],
                            acc.at[pl.ds(base + nfull * CH, rem)])
        plsc.subcore_barrier()

        bufs = (rows_0, rows_1)
        sems = (sem0, sem1)
        dsems = (semd0, semd1)

        def start_g(s, k):
            pltpu.async_copy(hrel_hbm.at[idx_v.at[s]], bufs[k], sems[k])
            pltpu.async_copy(dst_hbm.at[wid, s], dst_b.at[k], dsems[k])

        def wait_g(s, k):
            pltpu.make_async_copy(hrel_hbm.at[idx_v.at[s]], bufs[k],
                                  sems[k]).wait()
            pltpu.make_async_copy(dst_hbm.at[wid, s], dst_b.at[k],
                                  dsems[k]).wait()

        def scat(s, k):
            pltpu.sync_copy(bufs[k], acc.at[dst_b.at[k, 0]], add=True)

        npair = (NSTEPS - 1) // 2 if NSTEPS % 2 else (NSTEPS - 2) // 2
        start_g(0, 0)

        def pair(i, carry):
            s = 2 * i
            start_g(s + 1, 1)
            wait_g(s, 0)
            scat(s, 0)
            start_g(s + 2, 0)
            wait_g(s + 1, 1)
            scat(s + 1, 1)
            return carry
        lax.fori_loop(0, npair, pair, 0)
        if NSTEPS % 2:
            wait_g(NSTEPS - 1, 0)
            scat(NSTEPS - 1, 0)
        else:
            start_g(NSTEPS - 1, 1)
            wait_g(NSTEPS - 2, 0)
            scat(NSTEPS - 2, 0)
            wait_g(NSTEPS - 1, 1)
            scat(NSTEPS - 1, 1)
        plsc.subcore_barrier()

        pltpu.sync_copy(acc.at[pl.ds(base, RPT)],
                        out_hbm.at[cid, pl.ds(base, RPT)])

    return sc_body(hrel, gidxr, dstr)


def _edge_prep(src, et, dst, N, N_pad, E_pad):
    E = src.shape[0]
    W = 512
    rows, rows_p = E // W, E_pad // W
    extra = rows_p - rows
    npn = N_pad - N

    def body(s_ref, t_ref, d_ref, gi_ref, do_ref):
        z = jnp.zeros((extra, W), jnp.int32)
        s = jnp.concatenate([s_ref[...], z], 0)
        t = jnp.concatenate([t_ref[...], z], 0)
        dd = jnp.concatenate([d_ref[...], z], 0)
        f = (lax.broadcasted_iota(jnp.int32, (rows_p, W), 0) * W
             + lax.broadcasted_iota(jnp.int32, (rows_p, W), 1))
        real = f < E
        gi_ref[...] = jnp.where(real, t * N + s, (f - E) % N)
        do_ref[...] = jnp.where(real, dd, N + (f - E) % npn)

    return pl.pallas_call(
        body,
        out_shape=[
            jax.ShapeDtypeStruct((rows_p, W), jnp.int32),
            jax.ShapeDtypeStruct((rows_p, W), jnp.int32),
        ],
    )(src.reshape(rows, W), et.reshape(rows, W), dst.reshape(rows, W))


def _combine(parts, h, self_loop_weight):
    N, D = h.shape
    O = self_loop_weight.shape[1]
    BN = 2000

    def body(p_ref, h_ref, sw_ref, out_ref):
        out_ref[...] = (p_ref[0] + p_ref[1]
                        + jnp.dot(h_ref[...], sw_ref[...],
                                  preferred_element_type=jnp.float32))

    return pl.pallas_call(
        body,
        grid=(N // BN,),
        in_specs=[
            pl.BlockSpec((2, BN, O), lambda i: (0, i, 0)),
            pl.BlockSpec((BN, D), lambda i: (i, 0)),
            pl.BlockSpec((D, O), lambda i: (0, 0)),
        ],
        out_specs=pl.BlockSpec((BN, O), lambda i: (i, 0)),
        out_shape=jax.ShapeDtypeStruct((N, O), jnp.float32),
    )(parts, h, self_loop_weight)


def kernel(h, edge_index, edge_type, weight, w_comp, self_loop_weight):
    N, D = h.shape
    O = weight.shape[2]
    E = edge_type.shape[0]

    hrel = _hrel_table(h, weight, w_comp)

    NW = 32
    CH = 128
    T = -(-E // NW)
    T_pad = -(-T // CH) * CH
    pad = NW * T_pad - E
    NSTEPS = T_pad // CH
    src = edge_index[0].astype(jnp.int32)
    dst = edge_index[1].astype(jnp.int32)
    et = edge_type.astype(jnp.int32)
    N_pad = -(-(N + 1) // 128) * 128
    E_pad = NW * T_pad
    gidx, dstp = _edge_prep(src, et, dst, N, N_pad, E_pad)
    gidxr = gidx.reshape(NW, NSTEPS, CH)
    dstr = dstp.reshape(NW, NSTEPS, 1, CH)

    parts = _sc_gather_scatter(hrel, gidxr, dstr, N_pad, O)
    return _combine(parts, h, self_loop_weight)
